# baseline (speedup 1.0000x reference)
import functools

import jax
import jax.numpy as jnp
from jax import lax
from jax.experimental import pallas as pl
from jax.experimental.pallas import tpu as pltpu
from jax.experimental.pallas import tpu_sc as plsc

D = 64
L = 16
VROWS = 1000000
T2ROWS = 500032


def _wid():
    return lax.axis_index("s") * 2 + lax.axis_index("c")


@functools.cache
def _build_k1():
    n_blocks = (VROWS + 127) // 128
    per_w = n_blocks // 32
    extra = n_blocks - per_w * 32

    mesh = plsc.VectorSubcoreMesh(core_axis_name="c", subcore_axis_name="s")
    iota = lambda: lax.iota(jnp.int32, L)

    @functools.partial(
        pl.kernel,
        mesh=mesh,
        compiler_params=pltpu.CompilerParams(needs_layout_passes=False),
        out_type=jax.ShapeDtypeStruct((T2ROWS, 128), jnp.float32),
        scratch_types=[
            pltpu.VMEM((D, 128), jnp.float32),
            pltpu.VMEM((D, 128), jnp.float32),
            pltpu.SemaphoreType.DMA,
        ],
    )
    def k1(wt_hbm, t2_hbm, vin, vout, sem):
        w = _wid()

        def do_block(m, ncols):
            pltpu.async_copy(
                wt_hbm.at[:, pl.ds(m * 128, ncols)],
                vin.at[:, pl.ds(0, ncols)],
                sem,
            ).wait()
            nrows = ncols // 2
            for q in range(nrows * 128 // L):
                i0 = q * L
                r = i0 // 128
                rem = i0 % 128
                e = rem // D
                d0 = rem % D
                val = plsc.load_gather(
                    vin, [iota() + d0, jnp.full((L,), 2 * r + e, jnp.int32)]
                )
                vout[r, pl.ds(rem, L)] = val
            pltpu.async_copy(
                vout.at[pl.ds(0, nrows)],
                t2_hbm.at[pl.ds(m * D, nrows)],
                sem,
            ).wait()

        @pl.loop(0, per_w)
        def _(i):
            do_block(w * per_w + i, 128)

        @pl.when(w < extra)
        def _():
            do_block(32 * per_w + w, 128)

    return k1


@functools.cache
def _build_k23(BT: int, NB: int):
    KB = NB // 128
    n_slabs = BT * KB
    per_w = n_slabs // 32

    mesh = plsc.VectorSubcoreMesh(core_axis_name="c", subcore_axis_name="s")
    iota = lambda: lax.iota(jnp.int32, L)

    @functools.partial(
        pl.kernel,
        mesh=mesh,
        compiler_params=pltpu.CompilerParams(needs_layout_passes=False),
        out_type=jax.ShapeDtypeStruct((BT, D, NB), jnp.float32),
        scratch_types=[
            pltpu.VMEM((128,), jnp.int32),
            pltpu.VMEM((128,), jnp.int32),
            pltpu.VMEM((128, 128), jnp.float32),
            pltpu.VMEM((D, 128), jnp.float32),
            pltpu.SemaphoreType.DMA,
            pltpu.SemaphoreType.DMA,
        ],
    )
    def k23(tid_hbm, t2_hbm, out_hbm, vidx, vp, chunk, vslab, isem, gsem):
        w = _wid()

        @pl.loop(0, per_w)
        def _(s):
            sid = w * per_w + s
            t = sid // KB
            k = sid % KB
            pltpu.async_copy(
                tid_hbm.at[t, pl.ds(k * 128, 128)], vidx, isem
            ).wait()
            cbases = []
            for j8 in range(8):
                v = vidx[pl.ds(j8 * L, L)]
                vp[pl.ds(j8 * L, L)] = v >> 1
                cbases.append((v & 1) << 6)
            pltpu.async_copy(t2_hbm.at[vp], chunk, gsem).wait()
            for j8 in range(8):
                jrow = iota() + j8 * L
                cb = cbases[j8]
                for d in range(D):
                    val = plsc.load_gather(chunk, [jrow, cb + d])
                    vslab[d, pl.ds(j8 * L, L)] = val
            pltpu.async_copy(
                vslab, out_hbm.at[t, :, pl.ds(k * 128, 128)], gsem
            ).wait()

    return k23


def kernel(token_ids, w):
    BT = token_ids.shape[1]
    NB = token_ids.shape[0]
    wt = w.T
    tid_t = token_ids.T.astype(jnp.int32)
    table2 = _build_k1()(wt)
    out_t = _build_k23(BT, NB)(tid_t, table2)
    return out_t.transpose(2, 0, 1)


# speedup vs baseline: 1.0227x; 1.0227x over previous
import functools

import jax
import jax.numpy as jnp
from jax import lax
from jax.experimental import pallas as pl
from jax.experimental.pallas import tpu as pltpu
from jax.experimental.pallas import tpu_sc as plsc

D = 64
L = 16
VROWS = 1000000
T2ROWS = 500032


def _wid():
    return lax.axis_index("s") * 2 + lax.axis_index("c")


@functools.cache
def _build_k1():
    n_blocks = (VROWS + 127) // 128
    per_w = n_blocks // 32
    extra = n_blocks - per_w * 32

    mesh = plsc.VectorSubcoreMesh(core_axis_name="c", subcore_axis_name="s")
    iota = lambda: lax.iota(jnp.int32, L)

    @functools.partial(
        pl.kernel,
        mesh=mesh,
        compiler_params=pltpu.CompilerParams(needs_layout_passes=False),
        out_type=jax.ShapeDtypeStruct((T2ROWS, 128), jnp.float32),
        scratch_types=[
            pltpu.VMEM((D, 129), jnp.float32),
            pltpu.VMEM((D, 128), jnp.float32),
            pltpu.SemaphoreType.DMA,
        ],
    )
    def k1(wt_hbm, t2_hbm, vin, vout, sem):
        w = _wid()

        iot = iota()
        zero = jnp.zeros((L,), jnp.int32)

        def do_block(m, ncols):
            pltpu.async_copy(
                wt_hbm.at[:, pl.ds(m * 128, ncols)],
                vin.at[:, pl.ds(0, ncols)],
                sem,
            ).wait()
            nrows = ncols // 2

            @pl.loop(0, nrows * 128 // L, unroll=4)
            def _(q):
                i0 = q * L
                r = i0 >> 7
                rem = i0 & 127
                d0 = rem & (D - 1)
                j = 2 * r + (rem >> 6)
                val = plsc.load_gather(vin, [iot + d0, zero + j])
                vout[r, pl.ds(rem, L)] = val

            pltpu.async_copy(
                vout.at[pl.ds(0, nrows)],
                t2_hbm.at[pl.ds(m * D, nrows)],
                sem,
            ).wait()

        @pl.loop(0, per_w)
        def _(i):
            do_block(w * per_w + i, 128)

        @pl.when(w < extra)
        def _():
            do_block(32 * per_w + w, 128)

    return k1


@functools.cache
def _build_k23(BT: int, NB: int):
    KB = NB // 128
    n_slabs = BT * KB
    per_w = n_slabs // 32

    mesh = plsc.VectorSubcoreMesh(core_axis_name="c", subcore_axis_name="s")
    iota = lambda: lax.iota(jnp.int32, L)

    @functools.partial(
        pl.kernel,
        mesh=mesh,
        compiler_params=pltpu.CompilerParams(needs_layout_passes=False),
        out_type=jax.ShapeDtypeStruct((BT, D, NB), jnp.float32),
        scratch_types=[
            pltpu.VMEM((128,), jnp.int32),
            pltpu.VMEM((128,), jnp.int32),
            pltpu.VMEM((128, 129), jnp.float32),
            pltpu.VMEM((D, 128), jnp.float32),
            pltpu.SemaphoreType.DMA,
            pltpu.SemaphoreType.DMA,
        ],
    )
    def k23(tid_hbm, t2_hbm, out_hbm, vidx, vp, chunk, vslab, isem, gsem):
        w = _wid()
        iot = iota()
        jrows = [iot + j8 * L for j8 in range(8)]

        @pl.loop(0, per_w)
        def _(s):
            sid = w * per_w + s
            t = sid // KB
            k = sid % KB
            pltpu.async_copy(
                tid_hbm.at[t, pl.ds(k * 128, 128)], vidx, isem
            ).wait()
            cbases = []
            for j8 in range(8):
                v = vidx[pl.ds(j8 * L, L)]
                vp[pl.ds(j8 * L, L)] = v >> 1
                cbases.append((v & 1) << 6)
            pltpu.async_copy(t2_hbm.at[vp], chunk.at[:, pl.ds(0, 128)], gsem).wait()

            @pl.loop(0, D, unroll=4)
            def _(d):
                for j8 in range(8):
                    val = plsc.load_gather(chunk, [jrows[j8], cbases[j8] + d])
                    vslab[d, pl.ds(j8 * L, L)] = val

            pltpu.async_copy(
                vslab, out_hbm.at[t, :, pl.ds(k * 128, 128)], gsem
            ).wait()

    return k23


def kernel(token_ids, w):
    BT = token_ids.shape[1]
    NB = token_ids.shape[0]
    wt = w.T
    tid_t = token_ids.T.astype(jnp.int32)
    table2 = _build_k1()(wt)
    out_t = _build_k23(BT, NB)(tid_t, table2)
    return out_t.transpose(2, 0, 1)


# speedup vs baseline: 1.4635x; 1.4310x over previous
import functools

import jax
import jax.numpy as jnp
from jax import lax
from jax.experimental import pallas as pl
from jax.experimental.pallas import tpu as pltpu
from jax.experimental.pallas import tpu_sc as plsc

D = 64
L = 16
VROWS = 1000000
T2ROWS = 500032


def _wid():
    return lax.axis_index("s") * 2 + lax.axis_index("c")


@functools.cache
def _build_k1():
    n_blocks = (VROWS + 127) // 128
    per_w = n_blocks // 32
    extra = n_blocks - per_w * 32

    mesh = plsc.VectorSubcoreMesh(core_axis_name="c", subcore_axis_name="s")
    iota = lambda: lax.iota(jnp.int32, L)

    @functools.partial(
        pl.kernel,
        mesh=mesh,
        compiler_params=pltpu.CompilerParams(needs_layout_passes=False),
        out_type=jax.ShapeDtypeStruct((T2ROWS, 128), jnp.float32),
        scratch_types=[
            pltpu.VMEM((D, 129), jnp.float32),
            pltpu.VMEM((D, 128), jnp.float32),
            pltpu.SemaphoreType.DMA,
        ],
    )
    def k1(wt_hbm, t2_hbm, vin, vout, sem):
        w = _wid()

        iot = iota()
        zero = jnp.zeros((L,), jnp.int32)

        def do_block(m, ncols):
            pltpu.async_copy(
                wt_hbm.at[:, pl.ds(m * 128, ncols)],
                vin.at[:, pl.ds(0, ncols)],
                sem,
            ).wait()
            nrows = ncols // 2

            @plsc.parallel_loop(0, nrows * 128 // L, unroll=8)
            def _(q):
                i0 = q * L
                r = i0 >> 7
                rem = i0 & 127
                d0 = rem & (D - 1)
                j = 2 * r + (rem >> 6)
                val = plsc.load_gather(vin, [iot + d0, zero + j])
                vout[r, pl.ds(rem, L)] = val

            pltpu.async_copy(
                vout.at[pl.ds(0, nrows)],
                t2_hbm.at[pl.ds(m * D, nrows)],
                sem,
            ).wait()

        @pl.loop(0, per_w)
        def _(i):
            do_block(w * per_w + i, 128)

        @pl.when(w < extra)
        def _():
            do_block(32 * per_w + w, 128)

    return k1


@functools.cache
def _build_k23(BT: int, NB: int):
    KB = NB // 128
    n_slabs = BT * KB
    per_w = n_slabs // 32

    mesh = plsc.VectorSubcoreMesh(core_axis_name="c", subcore_axis_name="s")
    iota = lambda: lax.iota(jnp.int32, L)

    @functools.partial(
        pl.kernel,
        mesh=mesh,
        compiler_params=pltpu.CompilerParams(needs_layout_passes=False),
        out_type=jax.ShapeDtypeStruct((BT, D, NB), jnp.float32),
        scratch_types=[
            pltpu.VMEM((128,), jnp.int32),
            pltpu.VMEM((128,), jnp.int32),
            pltpu.VMEM((128, 129), jnp.float32),
            pltpu.VMEM((D, 128), jnp.float32),
            pltpu.SemaphoreType.DMA,
            pltpu.SemaphoreType.DMA,
        ],
    )
    def k23(tid_hbm, t2_hbm, out_hbm, vidx, vp, chunk, vslab, isem, gsem):
        w = _wid()
        iot = iota()
        jrows = [iot + j8 * L for j8 in range(8)]

        @pl.loop(0, per_w)
        def _(s):
            sid = w * per_w + s
            t = sid // KB
            k = sid % KB
            pltpu.async_copy(
                tid_hbm.at[t, pl.ds(k * 128, 128)], vidx, isem
            ).wait()
            cbases = []
            for j8 in range(8):
                v = vidx[pl.ds(j8 * L, L)]
                vp[pl.ds(j8 * L, L)] = v >> 1
                cbases.append((v & 1) << 6)
            pltpu.async_copy(t2_hbm.at[vp], chunk.at[:, pl.ds(0, 128)], gsem).wait()

            @plsc.parallel_loop(0, D, unroll=4)
            def _(d):
                for j8 in range(8):
                    val = plsc.load_gather(chunk, [jrows[j8], cbases[j8] + d])
                    vslab[d, pl.ds(j8 * L, L)] = val

            pltpu.async_copy(
                vslab, out_hbm.at[t, :, pl.ds(k * 128, 128)], gsem
            ).wait()

    return k23


def kernel(token_ids, w):
    BT = token_ids.shape[1]
    NB = token_ids.shape[0]
    wt = w.T
    tid_t = token_ids.T.astype(jnp.int32)
    table2 = _build_k1()(wt)
    out_t = _build_k23(BT, NB)(tid_t, table2)
    return out_t.transpose(2, 0, 1)


# speedup vs baseline: 1.6048x; 1.0966x over previous
import functools

import jax
import jax.numpy as jnp
from jax import lax
from jax.experimental import pallas as pl
from jax.experimental.pallas import tpu as pltpu
from jax.experimental.pallas import tpu_sc as plsc

D = 64
L = 16
VROWS = 1000000
T2ROWS = 500032


def _wid():
    return lax.axis_index("s") * 2 + lax.axis_index("c")


@functools.cache
def _build_k1():
    n_blocks = (VROWS + 127) // 128
    per_w = n_blocks // 32
    extra = n_blocks - per_w * 32

    mesh = plsc.VectorSubcoreMesh(core_axis_name="c", subcore_axis_name="s")
    iota = lambda: lax.iota(jnp.int32, L)

    @functools.partial(
        pl.kernel,
        mesh=mesh,
        compiler_params=pltpu.CompilerParams(needs_layout_passes=False),
        out_type=jax.ShapeDtypeStruct((T2ROWS, 128), jnp.float32),
        scratch_types=[
            pltpu.VMEM((D, 129), jnp.float32),
            pltpu.VMEM((D, 128), jnp.float32),
            pltpu.SemaphoreType.DMA,
        ],
    )
    def k1(wt_hbm, t2_hbm, vin, vout, sem):
        w = _wid()

        iot = iota()
        zero = jnp.zeros((L,), jnp.int32)
        iotd = [iot + d0 for d0 in range(0, D, L)]

        def do_block(m, ncols):
            pltpu.async_copy(
                wt_hbm.at[:, pl.ds(m * 128, ncols)],
                vin.at[:, pl.ds(0, ncols)],
                sem,
            ).wait()
            nrows = ncols // 2

            @plsc.parallel_loop(0, nrows, unroll=4)
            def _(r):
                j0 = zero + 2 * r
                j1 = j0 + 1
                for rem in range(0, 128, L):
                    d0 = rem & (D - 1)
                    jv = j0 if rem < D else j1
                    val = plsc.load_gather(vin, [iotd[d0 // L], jv])
                    vout[r, pl.ds(rem, L)] = val

            pltpu.async_copy(
                vout.at[pl.ds(0, nrows)],
                t2_hbm.at[pl.ds(m * D, nrows)],
                sem,
            ).wait()

        @pl.loop(0, per_w)
        def _(i):
            do_block(w * per_w + i, 128)

        @pl.when(w < extra)
        def _():
            do_block(32 * per_w + w, 128)

    return k1


@functools.cache
def _build_k23(BT: int, NB: int):
    KB = NB // 128
    n_slabs = BT * KB
    per_w = n_slabs // 32

    mesh = plsc.VectorSubcoreMesh(core_axis_name="c", subcore_axis_name="s")
    iota = lambda: lax.iota(jnp.int32, L)

    @functools.partial(
        pl.kernel,
        mesh=mesh,
        compiler_params=pltpu.CompilerParams(needs_layout_passes=False),
        out_type=jax.ShapeDtypeStruct((BT, D, NB), jnp.float32),
        scratch_types=[
            pltpu.VMEM((128,), jnp.int32),
            pltpu.VMEM((128,), jnp.int32),
            pltpu.VMEM((128, 129), jnp.float32),
            pltpu.VMEM((D, 128), jnp.float32),
            pltpu.SemaphoreType.DMA,
            pltpu.SemaphoreType.DMA,
        ],
    )
    def k23(tid_hbm, t2_hbm, out_hbm, vidx, vp, chunk, vslab, isem, gsem):
        w = _wid()
        iot = iota()
        jrows = [iot + j8 * L for j8 in range(8)]

        @pl.loop(0, per_w)
        def _(s):
            sid = w * per_w + s
            t = sid // KB
            k = sid % KB
            pltpu.async_copy(
                tid_hbm.at[t, pl.ds(k * 128, 128)], vidx, isem
            ).wait()
            cbases = []
            for j8 in range(8):
                v = vidx[pl.ds(j8 * L, L)]
                vp[pl.ds(j8 * L, L)] = v >> 1
                cbases.append((v & 1) << 6)
            pltpu.async_copy(t2_hbm.at[vp], chunk.at[:, pl.ds(0, 128)], gsem).wait()

            @plsc.parallel_loop(0, D, unroll=8)
            def _(d):
                for j8 in range(8):
                    val = plsc.load_gather(chunk, [jrows[j8], cbases[j8] + d])
                    vslab[d, pl.ds(j8 * L, L)] = val

            pltpu.async_copy(
                vslab, out_hbm.at[t, :, pl.ds(k * 128, 128)], gsem
            ).wait()

    return k23


def kernel(token_ids, w):
    BT = token_ids.shape[1]
    NB = token_ids.shape[0]
    wt = w.T
    tid_t = token_ids.T.astype(jnp.int32)
    table2 = _build_k1()(wt)
    out_t = _build_k23(BT, NB)(tid_t, table2)
    return out_t.transpose(2, 0, 1)


# speedup vs baseline: 2.8820x; 1.7958x over previous
import functools

import jax
import jax.numpy as jnp
from jax import lax
from jax.experimental import pallas as pl
from jax.experimental.pallas import tpu as pltpu
from jax.experimental.pallas import tpu_sc as plsc

EMBED_DIM = 64
IDX_PER_GATHER = 128
IB = 5
NBUF = 2


@functools.cache
def _build(B: int, D: int):
    info = plsc.get_sparse_core_info()
    NC, NS = info.num_cores, info.num_subcores
    NW = NC * NS
    rows_per_w = B // NW
    C = IB * IDX_PER_GATHER
    n_blocks = rows_per_w // C
    irows_per_w = rows_per_w // IDX_PER_GATHER
    assert rows_per_w % C == 0 and n_blocks % NBUF == 0
    assert B % (NW * IDX_PER_GATHER) == 0

    mesh = plsc.VectorSubcoreMesh(core_axis_name="c", subcore_axis_name="s")

    @functools.partial(
        pl.kernel,
        mesh=mesh,
        compiler_params=pltpu.CompilerParams(use_tc_tiling_on_sc=False),
        out_type=jax.ShapeDtypeStruct((B, D), jnp.float32),
        scratch_types=[
            pltpu.VMEM((irows_per_w, IDX_PER_GATHER), jnp.int32),
            pltpu.VMEM((NBUF, C, D), jnp.float32),
            pltpu.SemaphoreType.DMA((NBUF,)),
            pltpu.SemaphoreType.DMA((NBUF,)),
            pltpu.SemaphoreType.DMA,
        ],
    )
    def k(idx_hbm, table_hbm, out_hbm, idx_v, rows_v, gsem, osem, isem):
        wid = lax.axis_index("s") * NC + lax.axis_index("c")
        irow0 = wid * irows_per_w

        pltpu.async_copy(
            idx_hbm.at[pl.ds(irow0, irows_per_w)], idx_v, isem
        ).wait()

        def fire_g(g, s):
            for j in range(IB):
                pltpu.async_copy(
                    table_hbm.at[idx_v.at[g * IB + j]],
                    rows_v.at[s, pl.ds(j * IDX_PER_GATHER, IDX_PER_GATHER)],
                    gsem.at[s],
                )

        def drain_g(g, s):
            for j in range(IB):
                pltpu.make_async_copy(
                    table_hbm.at[idx_v.at[g * IB + j]],
                    rows_v.at[s, pl.ds(j * IDX_PER_GATHER, IDX_PER_GATHER)],
                    gsem.at[s],
                ).wait()

        def out_desc(g, s):
            return pltpu.make_async_copy(
                rows_v.at[s],
                out_hbm.at[pl.ds((irow0 + g * IB) * IDX_PER_GATHER, C)],
                osem.at[s],
            )

        fire_g(0, 0)
        fire_g(1, 1)
        drain_g(0, 0)
        out_desc(0, 0).start()

        @pl.loop(1, n_blocks - 1, step=NBUF)
        def _(g0):
            for b in range(NBUF):
                g = g0 + b
                s = (1 + b) % NBUF
                o = (s + 1) % NBUF
                out_desc_prev = pltpu.make_async_copy(
                    rows_v.at[o],
                    out_hbm.at[
                        pl.ds((irow0 + (g - 1) * IB) * IDX_PER_GATHER, C)
                    ],
                    osem.at[o],
                )
                out_desc_prev.wait()
                fire_g(g + 1, o)
                drain_g(g, s)
                out_desc(g, s).start()

        drain_g(n_blocks - 1, 1)
        out_desc(n_blocks - 1, 1).start()
        out_desc(n_blocks - 2, 0).wait()
        out_desc(n_blocks - 1, 1).wait()

    return k


def kernel(token_ids, w):
    B = token_ids.size
    D = w.shape[-1]
    idx2d = token_ids.reshape(B // IDX_PER_GATHER, IDX_PER_GATHER).astype(jnp.int32)
    out = _build(B, D)(idx2d, w)
    return out.reshape(token_ids.shape + (D,))
